# Initial kernel scaffold; baseline (speedup 1.0000x reference)
import functools

import jax
import jax.numpy as jnp
from jax import lax
from jax.experimental import pallas as pl
from jax.experimental.pallas import tpu as pltpu
from jax.experimental.pallas import tpu_sc as plsc

_DIM = 32
_LANG = 2
_BOS = 1
_NC = 2
_NS = 16
_LANES = 16
_NW = _NC * _NS


@functools.partial(jax.jit, static_argnums=(0, 1))
def _run(N, C, flat_tokens, table):
    PW = N // _NW
    G = PW // C
    mesh = plsc.VectorSubcoreMesh(core_axis_name="c", subcore_axis_name="s")

    @functools.partial(
        pl.kernel,
        out_type=jax.ShapeDtypeStruct((N, _DIM), jnp.float32),
        mesh=mesh,
        scratch_types=[
            pltpu.VMEM((C,), jnp.int32),
            pltpu.VMEM((C, _DIM), jnp.float32),
            pltpu.VMEM((8, _DIM), jnp.float32),
            pltpu.SemaphoreType.DMA,
        ],
    )
    def body(tokens_hbm, table_hbm, out_hbm, idx_v, rows_v, head_v, sem):
        wid = lax.axis_index("s") * _NC + lax.axis_index("c")
        base = wid * PW
        pltpu.sync_copy(table_hbm.at[pl.ds(0, 8)], head_v)
        lang_lo = head_v[_LANG, pl.ds(0, _LANES)]
        lang_hi = head_v[_LANG, pl.ds(_LANES, _LANES)]

        @pl.loop(0, G)
        def _chunk(g):
            off = base + g * C
            pltpu.sync_copy(tokens_hbm.at[pl.ds(off, C)], idx_v)

            @pl.loop(0, C // _LANES)
            def _fix(k):
                s = k * _LANES
                v = idx_v[pl.ds(s, _LANES)]
                idx_v[pl.ds(s, _LANES)] = jnp.where(v == _LANG, _BOS, v)

            pltpu.async_copy(table_hbm.at[idx_v], rows_v, sem).wait()

            @pl.loop(0, C)
            def _addlang(i):
                plsc.addupdate(rows_v.at[i, pl.ds(0, _LANES)], lang_lo)
                plsc.addupdate(rows_v.at[i, pl.ds(_LANES, _LANES)], lang_hi)

            pltpu.sync_copy(rows_v, out_hbm.at[pl.ds(off, C)])

    return body(flat_tokens, table)


def kernel(tokens, table):
    B, L = tokens.shape
    N = B * L
    out = _run(N, 1024, tokens.reshape(N), table)
    return out.reshape(B, L, _DIM)


# speedup vs baseline: 1.2824x; 1.2824x over previous
import functools

import jax
import jax.numpy as jnp
from jax import lax
from jax.experimental import pallas as pl
from jax.experimental.pallas import tpu as pltpu
from jax.experimental.pallas import tpu_sc as plsc

_DIM = 32
_LANG = 2
_BOS = 1
_NC = 2
_NS = 16
_LANES = 16
_NW = _NC * _NS


@functools.partial(jax.jit, static_argnums=(0, 1))
def _run(N, C, flat_tokens, table):
    PW = N // _NW
    G = PW // C
    mesh = plsc.VectorSubcoreMesh(core_axis_name="c", subcore_axis_name="s")

    @functools.partial(
        pl.kernel,
        out_type=jax.ShapeDtypeStruct((N, _DIM), jnp.float32),
        mesh=mesh,
        scratch_types=[
            pltpu.VMEM((C,), jnp.int32),
            pltpu.VMEM((C, _DIM), jnp.float32),
            pltpu.VMEM((8, _DIM), jnp.float32),
            pltpu.SemaphoreType.DMA,
        ],
        compiler_params=pltpu.CompilerParams(use_tc_tiling_on_sc=False),
    )
    def body(tokens_hbm, table_hbm, out_hbm, idx_v, rows_v, head_v, sem):
        wid = lax.axis_index("s") * _NC + lax.axis_index("c")
        base = wid * PW
        pltpu.sync_copy(table_hbm.at[pl.ds(0, 8)], head_v)
        lang_lo = head_v[_LANG, pl.ds(0, _LANES)]
        lang_hi = head_v[_LANG, pl.ds(_LANES, _LANES)]

        @pl.loop(0, G)
        def _chunk(g):
            off = base + g * C
            pltpu.sync_copy(tokens_hbm.at[pl.ds(off, C)], idx_v)

            @pl.loop(0, C // _LANES)
            def _fix(k):
                s = k * _LANES
                v = idx_v[pl.ds(s, _LANES)]
                idx_v[pl.ds(s, _LANES)] = jnp.where(v == _LANG, _BOS, v)

            pltpu.async_copy(table_hbm.at[idx_v], rows_v, sem).wait()

            @pl.loop(0, C)
            def _addlang(i):
                plsc.addupdate(rows_v.at[i, pl.ds(0, _LANES)], lang_lo)
                plsc.addupdate(rows_v.at[i, pl.ds(_LANES, _LANES)], lang_hi)

            pltpu.sync_copy(rows_v, out_hbm.at[pl.ds(off, C)])

    return body(flat_tokens, table)


def kernel(tokens, table):
    B, L = tokens.shape
    N = B * L
    out = _run(N, 1024, tokens.reshape(N), table)
    return out.reshape(B, L, _DIM)


# speedup vs baseline: 1.4685x; 1.1452x over previous
import functools

import jax
import jax.numpy as jnp
from jax import lax
from jax.experimental import pallas as pl
from jax.experimental.pallas import tpu as pltpu
from jax.experimental.pallas import tpu_sc as plsc

_DIM = 32
_LANG = 2
_BOS = 1
_NC = 2
_NS = 16
_LANES = 16
_NW = _NC * _NS


@functools.partial(jax.jit, static_argnums=(0, 1))
def _run(N, C, flat_tokens, table):
    PW = N // _NW
    G = PW // C
    mesh = plsc.VectorSubcoreMesh(core_axis_name="c", subcore_axis_name="s")

    @functools.partial(
        pl.kernel,
        out_type=jax.ShapeDtypeStruct((N, _DIM), jnp.float32),
        mesh=mesh,
        scratch_types=[
            pltpu.VMEM((2, C), jnp.int32),
            pltpu.VMEM((2, C, _DIM), jnp.float32),
            pltpu.VMEM((8, _DIM), jnp.float32),
            pltpu.SemaphoreType.DMA((2,)),
            pltpu.SemaphoreType.DMA((2,)),
        ],
        compiler_params=pltpu.CompilerParams(use_tc_tiling_on_sc=False),
    )
    def body(tokens_hbm, table_hbm, out_hbm, idx_v, rows_v, head_v, gsem, osem):
        wid = lax.axis_index("s") * _NC + lax.axis_index("c")
        base = wid * PW
        pltpu.sync_copy(table_hbm.at[pl.ds(0, 8)], head_v)
        lang_lo = head_v[_LANG, pl.ds(0, _LANES)]
        lang_hi = head_v[_LANG, pl.ds(_LANES, _LANES)]

        def stage_and_gather(g, b):
            off = base + g * C
            pltpu.sync_copy(tokens_hbm.at[pl.ds(off, C)], idx_v.at[b])

            @pl.loop(0, C // _LANES, unroll=8)
            def _fix(k):
                s = k * _LANES
                v = idx_v[b, pl.ds(s, _LANES)]
                idx_v[b, pl.ds(s, _LANES)] = jnp.where(v == _LANG, _BOS, v)

            pltpu.make_async_copy(
                table_hbm.at[idx_v.at[b]], rows_v.at[b], gsem.at[b]).start()

        stage_and_gather(0, 0)

        @pl.loop(0, G, step=2)
        def _pair(g0):
            for b in (0, 1):
                g = g0 + b
                nb = 1 - b
                pltpu.make_async_copy(
                    table_hbm.at[idx_v.at[b]], rows_v.at[b], gsem.at[b]).wait()

                @pl.when(g + 1 < G)
                def _():
                    @pl.when(g >= 1)
                    def _():
                        pltpu.make_async_copy(
                            rows_v.at[nb], out_hbm.at[pl.ds(base, C)],
                            osem.at[nb]).wait()

                    stage_and_gather(g + 1, nb)

                @pl.loop(0, C, unroll=8)
                def _addlang(i):
                    plsc.addupdate(rows_v.at[b, i, pl.ds(0, _LANES)], lang_lo)
                    plsc.addupdate(rows_v.at[b, i, pl.ds(_LANES, _LANES)],
                                   lang_hi)

                off = base + g * C
                pltpu.make_async_copy(
                    rows_v.at[b], out_hbm.at[pl.ds(off, C)], osem.at[b]).start()

        pltpu.make_async_copy(
            rows_v.at[0], out_hbm.at[pl.ds(base, C)], osem.at[0]).wait()
        pltpu.make_async_copy(
            rows_v.at[1], out_hbm.at[pl.ds(base, C)], osem.at[1]).wait()

    return body(flat_tokens, table)


def kernel(tokens, table):
    B, L = tokens.shape
    N = B * L
    out = _run(N, 1600, tokens.reshape(N), table)
    return out.reshape(B, L, _DIM)


# speedup vs baseline: 1.4725x; 1.0027x over previous
import functools

import jax
import jax.numpy as jnp
from jax import lax
from jax.experimental import pallas as pl
from jax.experimental.pallas import tpu as pltpu
from jax.experimental.pallas import tpu_sc as plsc

_DIM = 32
_LANG = 2
_BOS = 1
_NC = 2
_NS = 16
_LANES = 16
_NW = _NC * _NS
_R = 8


@functools.partial(jax.jit, static_argnums=(0, 1))
def _run(B, L, tokens, table):
    C = _R * L
    RW = B // _NW
    G = RW // _R
    mesh = plsc.VectorSubcoreMesh(core_axis_name="c", subcore_axis_name="s")

    starts = list(range(0, L - _LANES + 1, _LANES))
    if starts[-1] + _LANES < L:
        starts.append(L - _LANES)

    @functools.partial(
        pl.kernel,
        out_type=jax.ShapeDtypeStruct((B, L, _DIM), jnp.float32),
        mesh=mesh,
        scratch_types=[
            pltpu.VMEM((2, _R, L), jnp.int32),
            pltpu.VMEM((2, C), jnp.int32),
            pltpu.VMEM((2, C, _DIM), jnp.float32),
            pltpu.VMEM((8, _DIM), jnp.float32),
            pltpu.SemaphoreType.DMA((2,)),
            pltpu.SemaphoreType.DMA((2,)),
        ],
        compiler_params=pltpu.CompilerParams(use_tc_tiling_on_sc=False),
    )
    def body(tokens_hbm, table_hbm, out_hbm, tok_v, idx_v, rows_v, head_v,
             gsem, osem):
        wid = lax.axis_index("s") * _NC + lax.axis_index("c")
        row0 = wid * RW
        pltpu.sync_copy(table_hbm.at[pl.ds(0, 8)], head_v)
        lang_lo = head_v[_LANG, pl.ds(0, _LANES)]
        lang_hi = head_v[_LANG, pl.ds(_LANES, _LANES)]

        def stage_and_gather(g, b):
            r0 = row0 + g * _R
            pltpu.sync_copy(tokens_hbm.at[pl.ds(r0, _R)], tok_v.at[b])
            for r in range(_R):
                for s in starts:
                    v = tok_v[b, r, pl.ds(s, _LANES)]
                    idx_v[b, pl.ds(r * L + s, _LANES)] = jnp.where(
                        v == _LANG, _BOS, v)
            pltpu.make_async_copy(
                table_hbm.at[idx_v.at[b]], rows_v.at[b], gsem.at[b]).start()

        stage_and_gather(0, 0)

        @pl.loop(0, G, step=2)
        def _pair(g0):
            for b in (0, 1):
                g = g0 + b
                nb = 1 - b
                pltpu.make_async_copy(
                    table_hbm.at[idx_v.at[b]], rows_v.at[b], gsem.at[b]).wait()

                @pl.when(g + 1 < G)
                def _():
                    @pl.when(g >= 1)
                    def _():
                        for r in range(_R):
                            pltpu.make_async_copy(
                                rows_v.at[nb, pl.ds(r * L, L)],
                                out_hbm.at[row0 + r], osem.at[nb]).wait()

                    stage_and_gather(g + 1, nb)

                @pl.loop(0, C, unroll=8)
                def _addlang(i):
                    plsc.addupdate(rows_v.at[b, i, pl.ds(0, _LANES)], lang_lo)
                    plsc.addupdate(rows_v.at[b, i, pl.ds(_LANES, _LANES)],
                                   lang_hi)

                r0 = row0 + g * _R
                for r in range(_R):
                    pltpu.make_async_copy(
                        rows_v.at[b, pl.ds(r * L, L)],
                        out_hbm.at[r0 + r], osem.at[b]).start()

        for b in (0, 1):
            for r in range(_R):
                pltpu.make_async_copy(
                    rows_v.at[b, pl.ds(r * L, L)],
                    out_hbm.at[row0 + r], osem.at[b]).wait()

    return body(tokens, table)


def kernel(tokens, table):
    B, L = tokens.shape
    return _run(B, L, tokens, table)
